# baseline (speedup 1.0000x reference)
import functools

import jax
import jax.numpy as jnp
from jax import lax
from jax.experimental import pallas as pl
from jax.experimental.pallas import tpu as pltpu
from jax.experimental.pallas import tpu_sc as plsc

_N, _D_IN, _D_CODE, _K = 16384, 256, 32, 8192
_TILE = 256
_KC = 4096


def _sum32(p):
    s = ((p[:, 0:8] + p[:, 8:16]) + p[:, 16:24]) + p[:, 24:32]
    t = s[:, 0:4] + s[:, 4:8]
    t = t[:, 0:2] + t[:, 2:4]
    return t[:, 0] + t[:, 1]


def _vq_tc_body(x_ref, w_ref, cb_ref, ze_ref, idx_ref, loss_ref):
    x = x_ref[...]
    w = w_ref[...]
    z = lax.dot_general(x.astype(jnp.bfloat16), w.astype(jnp.bfloat16),
                        (((1,), (1,)), ((), ())),
                        preferred_element_type=jnp.float32)
    ze_ref[...] = z
    z2 = _sum32(z * z)[:, None]
    best_b = jnp.full((_TILE,), jnp.inf, jnp.float32)
    best_d = jnp.full((_TILE,), jnp.inf, jnp.float32)
    best_i = jnp.full((_TILE,), 0, jnp.int32)
    for c0 in range(0, _K, _KC):
        cb = cb_ref[pl.ds(c0, _KC), :]
        c2 = _sum32(cb * cb)
        zc = lax.dot_general(z.astype(jnp.bfloat16), cb.astype(jnp.bfloat16),
                             (((1,), (1,)), ((), ())),
                             preferred_element_type=jnp.float32)
        dist = z2 - 2.0 * zc + c2[None, :]
        dmin = jnp.min(dist, axis=1)
        iota = lax.broadcasted_iota(jnp.int32, dist.shape, 1)
        amin = jnp.min(jnp.where(dist == dmin[:, None], iota, _KC), axis=1)
        upd = dmin < best_b
        best_b = jnp.where(upd, dmin.astype(jnp.bfloat16).astype(jnp.float32),
                           best_b)
        best_d = jnp.where(upd, dmin, best_d)
        best_i = jnp.where(upd, amin + c0, best_i)
    idx_ref[0, 0, :] = best_i
    loss_ref[0, 0, :] = best_d * (1.0 / _D_CODE)


def _vq_tc(x, w, codebook):
    g = _N // _TILE
    ze, idx3, loss3 = pl.pallas_call(
        _vq_tc_body,
        grid=(g,),
        in_specs=[
            pl.BlockSpec((_TILE, _D_IN), lambda i: (i, 0)),
            pl.BlockSpec((_D_CODE, _D_IN), lambda i: (0, 0)),
            pl.BlockSpec((_K, _D_CODE), lambda i: (0, 0)),
        ],
        out_specs=[
            pl.BlockSpec((_TILE, _D_CODE), lambda i: (i, 0)),
            pl.BlockSpec((1, 1, _TILE), lambda i: (i, 0, 0)),
            pl.BlockSpec((1, 1, _TILE), lambda i: (i, 0, 0)),
        ],
        out_shape=[
            jax.ShapeDtypeStruct((_N, _D_CODE), jnp.float32),
            jax.ShapeDtypeStruct((g, 1, _TILE), jnp.int32),
            jax.ShapeDtypeStruct((g, 1, _TILE), jnp.float32),
        ],
    )(x, w, codebook)
    return ze, idx3.reshape(_N), loss3.reshape(_N)


def _make_sc_gather():
    info = plsc.get_sparse_core_info()
    nw = info.num_cores * info.num_subcores
    b_per_w = _N // nw
    mesh = plsc.VectorSubcoreMesh(core_axis_name="c", subcore_axis_name="s")

    @functools.partial(
        pl.kernel, mesh=mesh,
        compiler_params=pltpu.CompilerParams(use_tc_tiling_on_sc=False),
        out_type=jax.ShapeDtypeStruct((_N, _D_CODE), jnp.float32),
        scratch_types=[
            pltpu.VMEM((_N // nw,), jnp.int32),
            pltpu.VMEM((_N // nw, _D_CODE), jnp.float32),
            pltpu.SemaphoreType.DMA,
        ],
    )
    def gather_k(cb_hbm, idx_hbm, out_hbm, idx_v, rows_v, sem):
        wid = lax.axis_index("s") * info.num_cores + lax.axis_index("c")
        base = wid * b_per_w
        pltpu.sync_copy(idx_hbm.at[pl.ds(base, b_per_w)], idx_v)
        pltpu.async_copy(cb_hbm.at[idx_v], rows_v, sem).wait()
        pltpu.sync_copy(rows_v, out_hbm.at[pl.ds(base, b_per_w)])

    return gather_k


def kernel(x, W, codebook):
    ze, idx, loss = _vq_tc(x, W, codebook)
    zq = _make_sc_gather()(codebook, idx)
    zq_st = ze + lax.stop_gradient(zq - ze)
    return (zq_st, idx, loss)


# speedup vs baseline: 1.5055x; 1.5055x over previous
import functools

import jax
import jax.numpy as jnp
from jax import lax
from jax.experimental import pallas as pl
from jax.experimental.pallas import tpu as pltpu
from jax.experimental.pallas import tpu_sc as plsc

_N, _D_IN, _D_CODE, _K = 16384, 256, 32, 8192
_TILE = 512
_KC = 4096


def _sum32(p):
    s = ((p[:, 0:8] + p[:, 8:16]) + p[:, 16:24]) + p[:, 24:32]
    t = s[:, 0:4] + s[:, 4:8]
    t = t[:, 0:2] + t[:, 2:4]
    return t[:, 0] + t[:, 1]


def _vq_tc_body(x_ref, w_ref, cb_ref, ze_ref, idx_ref, loss_ref):
    x = x_ref[...]
    w = w_ref[...]
    z = lax.dot_general(x.astype(jnp.bfloat16), w.astype(jnp.bfloat16),
                        (((1,), (1,)), ((), ())),
                        preferred_element_type=jnp.float32)
    ze_ref[...] = z
    z2 = _sum32(z * z)[:, None]
    best_b = jnp.full((_TILE,), jnp.inf, jnp.float32)
    best_d = jnp.full((_TILE,), jnp.inf, jnp.float32)
    best_i = jnp.full((_TILE,), 0, jnp.int32)
    for c0 in range(0, _K, _KC):
        cb = cb_ref[pl.ds(c0, _KC), :]
        c2 = _sum32(cb * cb)
        zc = lax.dot_general(z.astype(jnp.bfloat16), cb.astype(jnp.bfloat16),
                             (((1,), (1,)), ((), ())),
                             preferred_element_type=jnp.float32)
        dist = z2 - 2.0 * zc + c2[None, :]
        dmin = jnp.min(dist, axis=1)
        iota = lax.broadcasted_iota(jnp.int32, dist.shape, 1)
        amin = jnp.min(jnp.where(dist == dmin[:, None], iota, _KC), axis=1)
        upd = dmin < best_b
        best_b = jnp.where(upd, dmin.astype(jnp.bfloat16).astype(jnp.float32),
                           best_b)
        best_d = jnp.where(upd, dmin, best_d)
        best_i = jnp.where(upd, amin + c0, best_i)
    idx_ref[0, 0, :] = best_i
    loss_ref[0, 0, :] = best_d * (1.0 / _D_CODE)


def _vq_tc(x, w, codebook):
    g = _N // _TILE
    ze, idx3, loss3 = pl.pallas_call(
        _vq_tc_body,
        grid=(g,),
        in_specs=[
            pl.BlockSpec((_TILE, _D_IN), lambda i: (i, 0)),
            pl.BlockSpec((_D_CODE, _D_IN), lambda i: (0, 0)),
            pl.BlockSpec((_K, _D_CODE), lambda i: (0, 0)),
        ],
        out_specs=[
            pl.BlockSpec((_TILE, _D_CODE), lambda i: (i, 0)),
            pl.BlockSpec((1, 1, _TILE), lambda i: (i, 0, 0)),
            pl.BlockSpec((1, 1, _TILE), lambda i: (i, 0, 0)),
        ],
        out_shape=[
            jax.ShapeDtypeStruct((_N, _D_CODE), jnp.float32),
            jax.ShapeDtypeStruct((g, 1, _TILE), jnp.int32),
            jax.ShapeDtypeStruct((g, 1, _TILE), jnp.float32),
        ],
    )(x, w, codebook)
    return ze, idx3.reshape(_N), loss3.reshape(_N)


def _make_sc_gather():
    info = plsc.get_sparse_core_info()
    nw = info.num_cores * info.num_subcores
    b_per_w = _N // nw
    mesh = plsc.VectorSubcoreMesh(core_axis_name="c", subcore_axis_name="s")

    @functools.partial(
        pl.kernel, mesh=mesh,
        compiler_params=pltpu.CompilerParams(use_tc_tiling_on_sc=False),
        out_type=jax.ShapeDtypeStruct((_N, _D_CODE), jnp.float32),
        scratch_types=[
            pltpu.VMEM((_N // nw,), jnp.int32),
            pltpu.VMEM((_N // nw, _D_CODE), jnp.float32),
            pltpu.SemaphoreType.DMA,
        ],
    )
    def gather_k(cb_hbm, idx_hbm, out_hbm, idx_v, rows_v, sem):
        wid = lax.axis_index("s") * info.num_cores + lax.axis_index("c")
        base = wid * b_per_w
        pltpu.sync_copy(idx_hbm.at[pl.ds(base, b_per_w)], idx_v)
        pltpu.async_copy(cb_hbm.at[idx_v], rows_v, sem).wait()
        pltpu.sync_copy(rows_v, out_hbm.at[pl.ds(base, b_per_w)])

    return gather_k


def kernel(x, W, codebook):
    ze, idx, loss = _vq_tc(x, W, codebook)
    zq = _make_sc_gather()(codebook, idx)
    zq_st = ze + lax.stop_gradient(zq - ze)
    return (zq_st, idx, loss)


# speedup vs baseline: 1.9159x; 1.2726x over previous
import functools

import jax
import jax.numpy as jnp
from jax import lax
from jax.experimental import pallas as pl
from jax.experimental.pallas import tpu as pltpu
from jax.experimental.pallas import tpu_sc as plsc

_N, _D_IN, _D_CODE, _K = 16384, 256, 32, 8192
_TILE = 1024
_KC = 4096


def _sum32(p):
    s = ((p[:, 0:8] + p[:, 8:16]) + p[:, 16:24]) + p[:, 24:32]
    t = s[:, 0:4] + s[:, 4:8]
    t = t[:, 0:2] + t[:, 2:4]
    return t[:, 0] + t[:, 1]


def _vq_tc_body(x_ref, w_ref, cb_ref, ze_ref, idx_ref, loss_ref):
    x = x_ref[...]
    w = w_ref[...]
    z = lax.dot_general(x.astype(jnp.bfloat16), w.astype(jnp.bfloat16),
                        (((1,), (1,)), ((), ())),
                        preferred_element_type=jnp.float32)
    ze_ref[...] = z
    z2 = _sum32(z * z)[:, None]
    best_b = jnp.full((_TILE,), jnp.inf, jnp.float32)
    best_d = jnp.full((_TILE,), jnp.inf, jnp.float32)
    best_i = jnp.full((_TILE,), 0, jnp.int32)
    for c0 in range(0, _K, _KC):
        cb = cb_ref[pl.ds(c0, _KC), :]
        c2 = _sum32(cb * cb)
        zc = lax.dot_general(z.astype(jnp.bfloat16), cb.astype(jnp.bfloat16),
                             (((1,), (1,)), ((), ())),
                             preferred_element_type=jnp.float32)
        dist = z2 - 2.0 * zc + c2[None, :]
        dmin = jnp.min(dist, axis=1)
        iota = lax.broadcasted_iota(jnp.int32, dist.shape, 1)
        amin = jnp.min(jnp.where(dist == dmin[:, None], iota, _KC), axis=1)
        upd = dmin < best_b
        best_b = jnp.where(upd, dmin.astype(jnp.bfloat16).astype(jnp.float32),
                           best_b)
        best_d = jnp.where(upd, dmin, best_d)
        best_i = jnp.where(upd, amin + c0, best_i)
    idx_ref[0, 0, :] = best_i
    loss_ref[0, 0, :] = best_d * (1.0 / _D_CODE)


def _vq_tc(x, w, codebook):
    g = _N // _TILE
    ze, idx3, loss3 = pl.pallas_call(
        _vq_tc_body,
        grid=(g,),
        in_specs=[
            pl.BlockSpec((_TILE, _D_IN), lambda i: (i, 0)),
            pl.BlockSpec((_D_CODE, _D_IN), lambda i: (0, 0)),
            pl.BlockSpec((_K, _D_CODE), lambda i: (0, 0)),
        ],
        out_specs=[
            pl.BlockSpec((_TILE, _D_CODE), lambda i: (i, 0)),
            pl.BlockSpec((1, 1, _TILE), lambda i: (i, 0, 0)),
            pl.BlockSpec((1, 1, _TILE), lambda i: (i, 0, 0)),
        ],
        out_shape=[
            jax.ShapeDtypeStruct((_N, _D_CODE), jnp.float32),
            jax.ShapeDtypeStruct((g, 1, _TILE), jnp.int32),
            jax.ShapeDtypeStruct((g, 1, _TILE), jnp.float32),
        ],
    )(x, w, codebook)
    return ze, idx3.reshape(_N), loss3.reshape(_N)


def _make_sc_gather():
    info = plsc.get_sparse_core_info()
    nw = info.num_cores * info.num_subcores
    b_per_w = _N // nw
    mesh = plsc.VectorSubcoreMesh(core_axis_name="c", subcore_axis_name="s")

    @functools.partial(
        pl.kernel, mesh=mesh,
        compiler_params=pltpu.CompilerParams(use_tc_tiling_on_sc=False),
        out_type=jax.ShapeDtypeStruct((_N, _D_CODE), jnp.float32),
        scratch_types=[
            pltpu.VMEM((_N // nw,), jnp.int32),
            pltpu.VMEM((_N // nw, _D_CODE), jnp.float32),
            pltpu.SemaphoreType.DMA,
        ],
    )
    def gather_k(cb_hbm, idx_hbm, out_hbm, idx_v, rows_v, sem):
        wid = lax.axis_index("s") * info.num_cores + lax.axis_index("c")
        base = wid * b_per_w
        pltpu.sync_copy(idx_hbm.at[pl.ds(base, b_per_w)], idx_v)
        pltpu.async_copy(cb_hbm.at[idx_v], rows_v, sem).wait()
        pltpu.sync_copy(rows_v, out_hbm.at[pl.ds(base, b_per_w)])

    return gather_k


def kernel(x, W, codebook):
    ze, idx, loss = _vq_tc(x, W, codebook)
    zq = _make_sc_gather()(codebook, idx)
    zq_st = ze + lax.stop_gradient(zq - ze)
    return (zq_st, idx, loss)


# speedup vs baseline: 2.1749x; 1.1352x over previous
import functools

import jax
import jax.numpy as jnp
from jax import lax
from jax.experimental import pallas as pl
from jax.experimental.pallas import tpu as pltpu
from jax.experimental.pallas import tpu_sc as plsc

_N, _D_IN, _D_CODE, _K = 16384, 256, 32, 8192
_TILE = 2048
_KC = 4096


def _sum32(p):
    s = ((p[:, 0:8] + p[:, 8:16]) + p[:, 16:24]) + p[:, 24:32]
    t = s[:, 0:4] + s[:, 4:8]
    t = t[:, 0:2] + t[:, 2:4]
    return t[:, 0] + t[:, 1]


def _vq_tc_body(x_ref, w_ref, cb_ref, ze_ref, idx_ref, loss_ref):
    x = x_ref[...]
    w = w_ref[...]
    z = lax.dot_general(x.astype(jnp.bfloat16), w.astype(jnp.bfloat16),
                        (((1,), (1,)), ((), ())),
                        preferred_element_type=jnp.float32)
    ze_ref[...] = z
    z2 = _sum32(z * z)[:, None]
    best_b = jnp.full((_TILE,), jnp.inf, jnp.float32)
    best_d = jnp.full((_TILE,), jnp.inf, jnp.float32)
    best_i = jnp.full((_TILE,), 0, jnp.int32)
    for c0 in range(0, _K, _KC):
        cb = cb_ref[pl.ds(c0, _KC), :]
        c2 = _sum32(cb * cb)
        zc = lax.dot_general(z.astype(jnp.bfloat16), cb.astype(jnp.bfloat16),
                             (((1,), (1,)), ((), ())),
                             preferred_element_type=jnp.float32)
        dist = z2 - 2.0 * zc + c2[None, :]
        dmin = jnp.min(dist, axis=1)
        iota = lax.broadcasted_iota(jnp.int32, dist.shape, 1)
        amin = jnp.min(jnp.where(dist == dmin[:, None], iota, _KC), axis=1)
        upd = dmin < best_b
        best_b = jnp.where(upd, dmin.astype(jnp.bfloat16).astype(jnp.float32),
                           best_b)
        best_d = jnp.where(upd, dmin, best_d)
        best_i = jnp.where(upd, amin + c0, best_i)
    idx_ref[0, 0, :] = best_i
    loss_ref[0, 0, :] = best_d * (1.0 / _D_CODE)


def _vq_tc(x, w, codebook):
    g = _N // _TILE
    ze, idx3, loss3 = pl.pallas_call(
        _vq_tc_body,
        grid=(g,),
        in_specs=[
            pl.BlockSpec((_TILE, _D_IN), lambda i: (i, 0)),
            pl.BlockSpec((_D_CODE, _D_IN), lambda i: (0, 0)),
            pl.BlockSpec((_K, _D_CODE), lambda i: (0, 0)),
        ],
        out_specs=[
            pl.BlockSpec((_TILE, _D_CODE), lambda i: (i, 0)),
            pl.BlockSpec((1, 1, _TILE), lambda i: (i, 0, 0)),
            pl.BlockSpec((1, 1, _TILE), lambda i: (i, 0, 0)),
        ],
        out_shape=[
            jax.ShapeDtypeStruct((_N, _D_CODE), jnp.float32),
            jax.ShapeDtypeStruct((g, 1, _TILE), jnp.int32),
            jax.ShapeDtypeStruct((g, 1, _TILE), jnp.float32),
        ],
    )(x, w, codebook)
    return ze, idx3.reshape(_N), loss3.reshape(_N)


def _make_sc_gather():
    info = plsc.get_sparse_core_info()
    nw = info.num_cores * info.num_subcores
    b_per_w = _N // nw
    mesh = plsc.VectorSubcoreMesh(core_axis_name="c", subcore_axis_name="s")

    @functools.partial(
        pl.kernel, mesh=mesh,
        compiler_params=pltpu.CompilerParams(use_tc_tiling_on_sc=False),
        out_type=jax.ShapeDtypeStruct((_N, _D_CODE), jnp.float32),
        scratch_types=[
            pltpu.VMEM((_N // nw,), jnp.int32),
            pltpu.VMEM((_N // nw, _D_CODE), jnp.float32),
            pltpu.SemaphoreType.DMA,
        ],
    )
    def gather_k(cb_hbm, idx_hbm, out_hbm, idx_v, rows_v, sem):
        wid = lax.axis_index("s") * info.num_cores + lax.axis_index("c")
        base = wid * b_per_w
        pltpu.sync_copy(idx_hbm.at[pl.ds(base, b_per_w)], idx_v)
        pltpu.async_copy(cb_hbm.at[idx_v], rows_v, sem).wait()
        pltpu.sync_copy(rows_v, out_hbm.at[pl.ds(base, b_per_w)])

    return gather_k


def kernel(x, W, codebook):
    ze, idx, loss = _vq_tc(x, W, codebook)
    zq = _make_sc_gather()(codebook, idx)
    zq_st = ze + lax.stop_gradient(zq - ze)
    return (zq_st, idx, loss)
